# baseline (speedup 1.0000x reference)
import jax
import jax.numpy as jnp
from jax.experimental import pallas as pl
from jax.experimental.pallas import tpu as pltpu

_WINDOW = 20
_EPS = 1e-08


def _swen_kernel(mag_ref, bias_ref, norm_ref, mean_ref, carry_ref):
    i = pl.program_id(0)
    F, B, TB = mag_ref.shape
    fs = jnp.sum(mag_ref[...], axis=0)

    @pl.when(i == 0)
    def _():
        carry_ref[...] = jnp.zeros_like(carry_ref)

    ext = jnp.concatenate([carry_ref[...], fs], axis=1)
    carry_ref[...] = ext[:, TB:]
    win = ext[:, 0:TB]
    for j in range(1, _WINDOW):
        win = win + ext[:, j:j + TB]

    t = i * TB + jax.lax.broadcasted_iota(jnp.int32, (B, TB), 1)
    count = jnp.minimum(t + 1, _WINDOW).astype(fs.dtype) * F
    mean = win / count + bias_ref[0]
    mean_ref[:, 0, :] = mean
    inv = 1.0 / (mean + _EPS)
    norm_ref[...] = mag_ref[...] * inv[None, :, :]


def kernel(mag, bias):
    B, F, T = mag.shape
    mag_t = jnp.transpose(mag, (1, 0, 2))
    TB = 896
    norm_t, mean = pl.pallas_call(
        _swen_kernel,
        grid=(pl.cdiv(T, TB),),
        in_specs=[
            pl.BlockSpec((F, B, TB), lambda i: (0, 0, i)),
            pl.BlockSpec(memory_space=pltpu.SMEM),
        ],
        out_specs=[
            pl.BlockSpec((F, B, TB), lambda i: (0, 0, i)),
            pl.BlockSpec((B, 1, TB), lambda i: (0, 0, i)),
        ],
        out_shape=[
            jax.ShapeDtypeStruct((F, B, T), mag.dtype),
            jax.ShapeDtypeStruct((B, 1, T), mag.dtype),
        ],
        scratch_shapes=[pltpu.VMEM((B, _WINDOW - 1), jnp.float32)],
        compiler_params=pltpu.CompilerParams(
            dimension_semantics=("arbitrary",),
            vmem_limit_bytes=100 * 1024 * 1024,
        ),
    )(mag_t, bias)
    return jnp.transpose(norm_t, (1, 0, 2)), mean


# speedup vs baseline: 1.0075x; 1.0075x over previous
import jax
import jax.numpy as jnp
from jax.experimental import pallas as pl
from jax.experimental.pallas import tpu as pltpu

_WINDOW = 20
_EPS = 1e-08


def _swen_kernel(mag_ref, bias_ref, norm_ref, mean_ref, carry_ref):
    i = pl.program_id(0)
    F, B, TB = mag_ref.shape
    fs = jnp.sum(mag_ref[...], axis=0)

    @pl.when(i == 0)
    def _():
        carry_ref[...] = jnp.zeros_like(carry_ref)

    ext = jnp.concatenate([carry_ref[...], fs], axis=1)
    carry_ref[...] = ext[:, TB:]
    win = ext[:, 0:TB]
    for j in range(1, _WINDOW):
        win = win + ext[:, j:j + TB]

    t = i * TB + jax.lax.broadcasted_iota(jnp.int32, (B, TB), 1)
    count = jnp.minimum(t + 1, _WINDOW).astype(fs.dtype) * F
    mean = win / count + bias_ref[0]
    mean_ref[:, 0, :] = mean
    inv = 1.0 / (mean + _EPS)
    norm_ref[...] = mag_ref[...] * inv[None, :, :]


def kernel(mag, bias):
    B, F, T = mag.shape
    mag_t = jnp.transpose(mag, (1, 0, 2))
    TB = 768
    norm_t, mean = pl.pallas_call(
        _swen_kernel,
        grid=(pl.cdiv(T, TB),),
        in_specs=[
            pl.BlockSpec((F, B, TB), lambda i: (0, 0, i)),
            pl.BlockSpec(memory_space=pltpu.SMEM),
        ],
        out_specs=[
            pl.BlockSpec((F, B, TB), lambda i: (0, 0, i)),
            pl.BlockSpec((B, 1, TB), lambda i: (0, 0, i)),
        ],
        out_shape=[
            jax.ShapeDtypeStruct((F, B, T), mag.dtype),
            jax.ShapeDtypeStruct((B, 1, T), mag.dtype),
        ],
        scratch_shapes=[pltpu.VMEM((B, _WINDOW - 1), jnp.float32)],
        compiler_params=pltpu.CompilerParams(
            dimension_semantics=("arbitrary",),
            vmem_limit_bytes=100 * 1024 * 1024,
        ),
    )(mag_t, bias)
    return jnp.transpose(norm_t, (1, 0, 2)), mean
